# Initial kernel scaffold; baseline (speedup 1.0000x reference)
import jax
import jax.numpy as jnp
from jax.experimental import pallas as pl


def kernel(D_output_fake, fake_data, real_data):
    raise NotImplementedError("write your pallas kernel here")


# speedup vs baseline: 261.8943x; 261.8943x over previous
import jax
import jax.numpy as jnp
from jax.experimental import pallas as pl

_ADV_W = 0.6
_NORM_W = 0.05
_DATA_W = 0.25


def _loss_kernel(d_ref, f_ref, r_ref, o_ref):
    d = d_ref[...]
    adv = -jnp.sum(jnp.log(d + 1e-08)) * (_ADV_W / d.size)
    diff = r_ref[...] - f_ref[...]
    sq = diff * diff
    n_each = sq.shape[0] * 3
    col = jax.lax.broadcasted_iota(jnp.int32, sq.shape, 1)
    w = jnp.where(col < 3, _DATA_W / n_each, _NORM_W / n_each)
    o_ref[...] = jnp.reshape(adv + jnp.sum(sq * w), (1, 1))


def kernel(D_output_fake, fake_data, real_data):
    f = fake_data.reshape(-1, 6)
    r = real_data.reshape(-1, 6)
    out = pl.pallas_call(
        _loss_kernel,
        out_shape=jax.ShapeDtypeStruct((1, 1), jnp.float32),
    )(D_output_fake, f, r)
    return out[0, 0]
